# baseline (speedup 1.0000x reference)
import jax
import jax.numpy as jnp
from jax import lax
from jax.experimental import pallas as pl
from jax.experimental.pallas import tpu as pltpu
from jax.experimental.pallas import tpu_sc as plsc

_G = 32
_NV = _G * _G * _G
_B = 4
_D = 128
_N = 100000

_L = 16
_S1C = 6256
_S1V = _S1C // _L
_IDX_LEN = 400192
_CH = 10000
_NCH = _N // _CH
_CHV = _CH // _L
_ACCV = _NV // _L


def _splat(val, dtype):
    return jnp.full((_L,), val, dtype)


def _sc_body(xyz_ref, feat_ref, idx_ref, vmax_ref, cnt_ref,
             xyzbuf, obuf, acc, cacc, fbuf, ibuf):
    c = lax.axis_index("c")
    s = lax.axis_index("s")
    lane = lax.iota(jnp.int32, _L)

    k31 = _splat(31, jnp.int32)
    k0 = _splat(0, jnp.int32)
    scale = _splat(32.0, jnp.float32)

    def _quant(u):
        ui = (u * scale).astype(jnp.int32)
        return jnp.minimum(jnp.maximum(ui, k0), k31)

    for t in range(2):
        off = (c * 200000 + (s + 16 * t) * _S1C).astype(jnp.int32)

        pltpu.sync_copy(xyz_ref.at[pl.ds(off * 3, _S1C * 3)], xyzbuf)

        def s1_body(v, carry):
            ix = v * 48 + lane * 3
            x = plsc.load_gather(xyzbuf, [ix])
            y = plsc.load_gather(xyzbuf, [ix + 1])
            z = plsc.load_gather(xyzbuf, [ix + 2])
            flat = (_quant(x) * _splat(_G * _G, jnp.int32)
                    + _quant(y) * _splat(_G, jnp.int32) + _quant(z))
            obuf[pl.ds(v * _L, _L)] = flat
            return carry

        lax.fori_loop(0, _S1V, s1_body, 0)
        pltpu.sync_copy(obuf, idx_ref.at[pl.ds(off, _S1C)])

    plsc.subcore_barrier()

    b = c * 2 + s // 8
    dbase = (s % 8) * 16
    idx_base = b * _N
    neginf = _splat(-jnp.inf, jnp.float32)
    zero = _splat(0.0, jnp.float32)

    def row(j, carry):
        r = b * _D + dbase + j

        def ini(i, cc):
            acc[pl.ds(i * _L, _L)] = neginf
            return cc

        lax.fori_loop(0, _ACCV, ini, 0)

        def chunk(ci, cc):
            pltpu.sync_copy(feat_ref.at[pl.ds(r * _N + ci * _CH, _CH)], fbuf)
            pltpu.sync_copy(idx_ref.at[pl.ds(idx_base + ci * _CH, _CH)], ibuf)

            def inner(i, c3):
                vals = fbuf[pl.ds(i * _L, _L)]
                vidx = ibuf[pl.ds(i * _L, _L)]
                cur = plsc.load_gather(acc, [vidx])
                plsc.store_scatter(acc, [vidx], jnp.maximum(cur, vals))
                chk = plsc.load_gather(acc, [vidx])
                nbad = jnp.sum((chk < vals).astype(jnp.int32))

                def fcond(nb):
                    return nb > 0

                def fbody(nb):
                    c2 = plsc.load_gather(acc, [vidx])
                    bad = c2 < vals
                    plsc.store_scatter(acc, [vidx], jnp.maximum(c2, vals),
                                       mask=bad)
                    c4 = plsc.load_gather(acc, [vidx])
                    return jnp.sum((c4 < vals).astype(jnp.int32))

                lax.while_loop(fcond, fbody, nbad)
                return c3

            lax.fori_loop(0, _CHV, inner, 0)
            return cc

        lax.fori_loop(0, _NCH, chunk, 0)

        def wb(i, cc):
            v = acc[pl.ds(i * _L, _L)]
            acc[pl.ds(i * _L, _L)] = jnp.where(v == neginf, zero, v)
            return cc

        lax.fori_loop(0, _ACCV, wb, 0)
        pltpu.sync_copy(acc, vmax_ref.at[pl.ds(r * _NV, _NV)])
        return carry

    lax.fori_loop(0, 16, row, 0)

    @pl.when(s % 8 == 0)
    def _counts():
        izero = _splat(0, jnp.int32)
        ones = _splat(1, jnp.int32)

        def ini(i, cc):
            cacc[pl.ds(i * _L, _L)] = izero
            return cc

        lax.fori_loop(0, _ACCV, ini, 0)

        def chunk(ci, cc):
            pltpu.sync_copy(idx_ref.at[pl.ds(idx_base + ci * _CH, _CH)], ibuf)

            def inner(i, c3):
                vidx = ibuf[pl.ds(i * _L, _L)]
                plsc.addupdate_scatter(cacc, [vidx], ones)
                return c3

            lax.fori_loop(0, _CHV, inner, 0)
            return cc

        lax.fori_loop(0, _NCH, chunk, 0)

        def fix(i, cc):
            v = cacc[pl.ds(i * _L, _L)]
            cacc[pl.ds(i * _L, _L)] = jnp.maximum(v, ones)
            return cc

        lax.fori_loop(0, _ACCV, fix, 0)
        pltpu.sync_copy(cacc, cnt_ref.at[pl.ds(b * _NV, _NV)])


_MESH = plsc.VectorSubcoreMesh(core_axis_name="c", subcore_axis_name="s")

_SC_CALL = pl.kernel(
    _sc_body,
    out_type=(
        jax.ShapeDtypeStruct((_IDX_LEN,), jnp.int32),
        jax.ShapeDtypeStruct((_B * _D * _NV,), jnp.float32),
        jax.ShapeDtypeStruct((_B * _NV,), jnp.int32),
    ),
    mesh=_MESH,
    compiler_params=pltpu.CompilerParams(needs_layout_passes=False),
    scratch_types=[
        pltpu.VMEM((_S1C * 3,), jnp.float32),
        pltpu.VMEM((_S1C,), jnp.int32),
        pltpu.VMEM((_NV,), jnp.float32),
        pltpu.VMEM((_NV,), jnp.int32),
        pltpu.VMEM((_CH,), jnp.float32),
        pltpu.VMEM((_CH,), jnp.int32),
    ],
)


@jax.jit
def kernel(features, xyz_coords_for_voxelization):
    f = features.reshape(-1)
    xyz = xyz_coords_for_voxelization.reshape(-1)
    xyz = jnp.pad(xyz, (0, _IDX_LEN * 3 - xyz.shape[0]))
    idxp, vmax, cnt = _SC_CALL(xyz, f)
    return (
        vmax.reshape(_B, _D, _G, _G, _G),
        idxp[: _B * _N].reshape(_B, _N),
        cnt.reshape(_B, 1, _NV),
    )


# speedup vs baseline: 1.7216x; 1.7216x over previous
import jax
import jax.numpy as jnp
from jax import lax
from jax.experimental import pallas as pl
from jax.experimental.pallas import tpu as pltpu
from jax.experimental.pallas import tpu_sc as plsc

_G = 32
_NV = _G * _G * _G
_B = 4
_D = 128
_N = 100000

_L = 16
_S1C = 6256
_S1V = _S1C // _L
_IDX_LEN = 400192
_CH = 10000
_NCH = _N // _CH
_CHV = _CH // _L
_ACCV = _NV // _L


def _splat(val, dtype):
    return jnp.full((_L,), val, dtype)


def _sc_body(xyz_ref, feat_ref, idx_ref, vmax_ref, cnt_ref,
             xyzbuf, ibuf, acc1, acc2, fbuf1):
    c = lax.axis_index("c")
    s = lax.axis_index("s")
    lane = lax.iota(jnp.int32, _L)

    k31 = _splat(31, jnp.int32)
    k0 = _splat(0, jnp.int32)
    scale = _splat(32.0, jnp.float32)

    def _quant(u):
        ui = (u * scale).astype(jnp.int32)
        return jnp.minimum(jnp.maximum(ui, k0), k31)

    for t in range(2):
        off = c * 200000 + (s + 16 * t) * _S1C

        pltpu.sync_copy(xyz_ref.at[pl.ds(off * 3, _S1C * 3)], xyzbuf)

        def s1_body(v, carry):
            ix = v * 48 + lane * 3
            x = plsc.load_gather(xyzbuf, [ix])
            y = plsc.load_gather(xyzbuf, [ix + 1])
            z = plsc.load_gather(xyzbuf, [ix + 2])
            flat = (_quant(x) * _splat(_G * _G, jnp.int32)
                    + _quant(y) * _splat(_G, jnp.int32) + _quant(z))
            ibuf[pl.ds(v * _L, _L)] = flat
            return carry

        lax.fori_loop(0, _S1V, s1_body, 0)
        pltpu.sync_copy(ibuf.at[pl.ds(0, _S1C)], idx_ref.at[pl.ds(off, _S1C)])

    plsc.subcore_barrier()

    b = c * 2 + s // 8
    dbase = (s % 8) * 16
    idx_base = b * _N
    neginf = _splat(-jnp.inf, jnp.float32)
    zero = _splat(0.0, jnp.float32)
    false_v = jnp.zeros((_L,), jnp.bool_)

    def _retry_max(accref, vidx, vals):
        cur = plsc.load_gather(accref, [vidx])
        plsc.store_scatter(accref, [vidx], jnp.maximum(cur, vals))
        chk = plsc.load_gather(accref, [vidx])
        nbad = jnp.sum((chk < vals).astype(jnp.int32))

        def fcond(nb):
            return nb > 0

        def fbody(nb):
            c2 = plsc.load_gather(accref, [vidx])
            plsc.store_scatter(accref, [vidx], jnp.maximum(c2, vals),
                               mask=c2 < vals)
            c4 = plsc.load_gather(accref, [vidx])
            return jnp.sum((c4 < vals).astype(jnp.int32))

        lax.while_loop(fcond, fbody, nbad)

    def pair(p, carry):
        r1 = b * _D + dbase + 2 * p
        r2 = r1 + 1

        def ini(i, cc):
            sl = pl.ds(i * _L, _L)
            acc1[sl] = neginf
            acc2[sl] = neginf
            return cc

        lax.fori_loop(0, _ACCV, ini, 0)

        def chunk(ci, cc):
            pltpu.sync_copy(feat_ref.at[pl.ds(r1 * _N + ci * _CH, _CH)],
                            fbuf1)
            pltpu.sync_copy(feat_ref.at[pl.ds(r2 * _N + ci * _CH, _CH)],
                            xyzbuf.at[pl.ds(0, _CH)])
            pltpu.sync_copy(idx_ref.at[pl.ds(idx_base + ci * _CH, _CH)],
                            ibuf.at[pl.ds(0, _CH)])

            def inner(i, bad):
                sl = pl.ds(i * _L, _L)
                ix = ibuf[sl]
                v1 = fbuf1[sl]
                v2 = xyzbuf[sl]
                c1 = plsc.load_gather(acc1, [ix])
                plsc.store_scatter(acc1, [ix], jnp.maximum(c1, v1))
                c2 = plsc.load_gather(acc2, [ix])
                plsc.store_scatter(acc2, [ix], jnp.maximum(c2, v2))
                k1 = plsc.load_gather(acc1, [ix])
                plsc.store_scatter(acc1, [ix], jnp.maximum(k1, v1),
                                   mask=k1 < v1)
                k2 = plsc.load_gather(acc2, [ix])
                plsc.store_scatter(acc2, [ix], jnp.maximum(k2, v2),
                                   mask=k2 < v2)
                w1 = plsc.load_gather(acc1, [ix])
                w2 = plsc.load_gather(acc2, [ix])
                return bad | (w1 < v1) | (w2 < v2)

            bad = lax.fori_loop(0, _CHV, inner, false_v)
            nbad = jnp.sum(bad.astype(jnp.int32))

            @pl.when(nbad > 0)
            def _redo():
                def redo(i, cc2):
                    sl = pl.ds(i * _L, _L)
                    ix = ibuf[sl]
                    _retry_max(acc1, ix, fbuf1[sl])
                    _retry_max(acc2, ix, xyzbuf[sl])
                    return cc2

                lax.fori_loop(0, _CHV, redo, 0)

            return cc

        lax.fori_loop(0, _NCH, chunk, 0)

        def wb(i, cc):
            sl = pl.ds(i * _L, _L)
            v1 = acc1[sl]
            acc1[sl] = jnp.where(v1 == neginf, zero, v1)
            v2 = acc2[sl]
            acc2[sl] = jnp.where(v2 == neginf, zero, v2)
            return cc

        lax.fori_loop(0, _ACCV, wb, 0)
        pltpu.sync_copy(acc1, vmax_ref.at[pl.ds(r1 * _NV, _NV)])
        pltpu.sync_copy(acc2, vmax_ref.at[pl.ds(r2 * _NV, _NV)])
        return carry

    lax.fori_loop(0, 8, pair, 0)

    @pl.when(s % 8 == 0)
    def _counts():
        ones = _splat(1.0, jnp.float32)

        def ini(i, cc):
            acc2[pl.ds(i * _L, _L)] = zero
            return cc

        lax.fori_loop(0, _ACCV, ini, 0)

        def chunk(ci, cc):
            pltpu.sync_copy(idx_ref.at[pl.ds(idx_base + ci * _CH, _CH)],
                            ibuf.at[pl.ds(0, _CH)])

            def inner(i, c3):
                vidx = ibuf[pl.ds(i * _L, _L)]
                plsc.addupdate_scatter(acc2, [vidx], ones)
                return c3

            lax.fori_loop(0, _CHV, inner, 0)
            return cc

        lax.fori_loop(0, _NCH, chunk, 0)

        def fix(i, cc):
            sl = pl.ds(i * _L, _L)
            acc2[sl] = jnp.maximum(acc2[sl], ones)
            return cc

        lax.fori_loop(0, _ACCV, fix, 0)
        pltpu.sync_copy(acc2, cnt_ref.at[pl.ds(b * _NV, _NV)])


_MESH = plsc.VectorSubcoreMesh(core_axis_name="c", subcore_axis_name="s")

_SC_CALL = pl.kernel(
    _sc_body,
    out_type=(
        jax.ShapeDtypeStruct((_IDX_LEN,), jnp.int32),
        jax.ShapeDtypeStruct((_B * _D * _NV,), jnp.float32),
        jax.ShapeDtypeStruct((_B * _NV,), jnp.float32),
    ),
    mesh=_MESH,
    compiler_params=pltpu.CompilerParams(needs_layout_passes=False),
    scratch_types=[
        pltpu.VMEM((_S1C * 3,), jnp.float32),
        pltpu.VMEM((_CH,), jnp.int32),
        pltpu.VMEM((_NV,), jnp.float32),
        pltpu.VMEM((_NV,), jnp.float32),
        pltpu.VMEM((_CH,), jnp.float32),
    ],
)


@jax.jit
def kernel(features, xyz_coords_for_voxelization):
    f = features.reshape(-1)
    xyz = xyz_coords_for_voxelization.reshape(-1)
    xyz = jnp.pad(xyz, (0, _IDX_LEN * 3 - xyz.shape[0]))
    idxp, vmax, cnt = _SC_CALL(xyz, f)
    return (
        vmax.reshape(_B, _D, _G, _G, _G),
        idxp[: _B * _N].reshape(_B, _N),
        cnt.astype(jnp.int32).reshape(_B, 1, _NV),
    )


# speedup vs baseline: 2.2587x; 1.3120x over previous
import jax
import jax.numpy as jnp
from jax import lax
from jax.experimental import pallas as pl
from jax.experimental.pallas import tpu as pltpu
from jax.experimental.pallas import tpu_sc as plsc

_G = 32
_NV = _G * _G * _G
_B = 4
_D = 128
_N = 100000

_L = 16
_S1C = 3200
_S1V = _S1C // _L
_IDX_LEN = 404800
_CH = 10000
_NCH = _N // _CH
_CHV = _CH // _L
_UNR = 5
_ACCV = _NV // _L


def _splat(val, dtype):
    return jnp.full((_L,), val, dtype)


def _sc_body(xyz_ref, feat_ref, idx_ref, vmax_ref, cnt_ref, wstr_ref,
             fb1a, fb1b, fb2a, fb2b, wba, wbb, acc1, acc2, tmp,
             sf1a, sf2a, swa, sf1b, sf2b, swb):
    c = lax.axis_index("c")
    s = lax.axis_index("s")
    lane = lax.iota(jnp.int32, _L)
    lanep = jnp.minimum(lane + 1, _splat(15, jnp.int32))
    lanem = jnp.maximum(lane - 1, _splat(0, jnp.int32))

    k31 = _splat(31, jnp.int32)
    k0 = _splat(0, jnp.int32)
    scale = _splat(32.0, jnp.float32)

    def _quant(u):
        ui = (u * scale).astype(jnp.int32)
        return jnp.minimum(jnp.maximum(ui, k0), k31)

    for t in range(4):
        off = c * 200000 + (s + 16 * t) * _S1C

        pltpu.sync_copy(xyz_ref.at[pl.ds(off * 3, _S1C * 3)],
                        fb1a.at[pl.ds(0, _S1C * 3)])

        def s1_body(g, carry):
            ix = g * 48 + lane * 3
            x = plsc.load_gather(fb1a, [ix])
            y = plsc.load_gather(fb1a, [ix + 1])
            z = plsc.load_gather(fb1a, [ix + 2])
            flat = (_quant(x) * _splat(_G * _G, jnp.int32)
                    + _quant(y) * _splat(_G, jnp.int32) + _quant(z))
            wba[pl.ds(g * _L, _L)] = flat
            sk, sperm = plsc.sort_key_val(flat, lane)
            tmp[pl.ds(0, _L)] = sk
            tmp[pl.ds(_L, _L)] = sperm
            skp = plsc.load_gather(tmp, [lanep])
            skm = plsc.load_gather(tmp, [lanem])
            spp = plsc.load_gather(tmp, [lanep + _L])
            spm = plsc.load_gather(tmp, [lanem + _L])
            nb = jnp.where(sk == skp, spp,
                           jnp.where(sk == skm, spm, sperm))
            word = sk + sperm * _splat(1 << 15, jnp.int32) \
                + nb * _splat(1 << 19, jnp.int32)
            wbb[pl.ds(g * _L, _L)] = word
            return carry

        lax.fori_loop(0, _S1V, s1_body, 0)
        pltpu.sync_copy(wba.at[pl.ds(0, _S1C)], idx_ref.at[pl.ds(off, _S1C)])
        pltpu.sync_copy(wbb.at[pl.ds(0, _S1C)], wstr_ref.at[pl.ds(off, _S1C)])

    plsc.subcore_barrier()

    b = c * 2 + s // 8
    dbase = (s % 8) * 16
    wbase = b * _N
    neginf = _splat(-jnp.inf, jnp.float32)
    zero = _splat(0.0, jnp.float32)
    false_v = jnp.zeros((_L,), jnp.bool_)
    m15 = _splat((1 << 15) - 1, jnp.int32)
    m4 = _splat(15, jnp.int32)

    def _decode(w):
        six = w & m15
        sp = (w >> _splat(15, jnp.int32)) & m4
        nb = (w >> _splat(19, jnp.int32)) & m4
        return six, sp, nb

    def _retry_max(accref, vidx, vals):
        cur = plsc.load_gather(accref, [vidx])
        plsc.store_scatter(accref, [vidx], jnp.maximum(cur, vals))
        chk = plsc.load_gather(accref, [vidx])
        nbad = jnp.sum((chk < vals).astype(jnp.int32))

        def fcond(nb_):
            return nb_ > 0

        def fbody(nb_):
            c2 = plsc.load_gather(accref, [vidx])
            plsc.store_scatter(accref, [vidx], jnp.maximum(c2, vals),
                               mask=c2 < vals)
            c4 = plsc.load_gather(accref, [vidx])
            return jnp.sum((c4 < vals).astype(jnp.int32))

        lax.while_loop(fcond, fbody, nbad)

    bufsets = ((fb1a, fb2a, (sf1a, sf2a, swa), wba),
               (fb1b, fb2b, (sf1b, sf2b, swb), wbb))

    def pair(p, carry):
        r1 = b * _D + dbase + 2 * p
        r2 = r1 + 1

        def _copies(ci, bufset):
            f1, f2, sems, wb_ = bufset
            return (
                pltpu.make_async_copy(
                    feat_ref.at[pl.ds(r1 * _N + ci * _CH, _CH)], f1, sems[0]),
                pltpu.make_async_copy(
                    feat_ref.at[pl.ds(r2 * _N + ci * _CH, _CH)], f2, sems[1]),
                pltpu.make_async_copy(
                    wstr_ref.at[pl.ds(wbase + ci * _CH, _CH)], wb_, sems[2]),
            )

        def _start(ci, bufset):
            for cp in _copies(ci, bufset):
                cp.start()

        def _wait(ci, bufset):
            for cp in _copies(ci, bufset):
                cp.wait()

        _start(0, bufsets[0])
        _start(1, bufsets[1])

        def ini(i, cc):
            for u in range(4):
                sl = pl.ds((i * 4 + u) * _L, _L)
                acc1[sl] = neginf
                acc2[sl] = neginf
            return cc

        lax.fori_loop(0, _ACCV // 4, ini, 0)

        def chunk_grp(g2, cc):
            for bi in range(2):
                f1, f2, sems, wb_ = bufsets[bi]
                ci = 2 * g2 + bi
                _wait(ci, bufsets[bi])

                def inner(i2, bad):
                    for u in range(_UNR):
                        i = i2 * _UNR + u
                        base = _splat(i * _L, jnp.int32)
                        w = wb_[pl.ds(i * _L, _L)]
                        six, sp, nb = _decode(w)
                        sp = sp + base
                        nb = nb + base
                        v1 = jnp.maximum(plsc.load_gather(f1, [sp]),
                                         plsc.load_gather(f1, [nb]))
                        v2 = jnp.maximum(plsc.load_gather(f2, [sp]),
                                         plsc.load_gather(f2, [nb]))
                        c1 = plsc.load_gather(acc1, [six])
                        plsc.store_scatter(acc1, [six], jnp.maximum(c1, v1))
                        c2 = plsc.load_gather(acc2, [six])
                        plsc.store_scatter(acc2, [six], jnp.maximum(c2, v2))
                        w1 = plsc.load_gather(acc1, [six])
                        w2 = plsc.load_gather(acc2, [six])
                        bad = bad | (w1 < v1) | (w2 < v2)
                    return bad

                bad = lax.fori_loop(0, _CHV // _UNR, inner, false_v)
                nbad = jnp.sum(bad.astype(jnp.int32))

                @pl.when(nbad > 0)
                def _redo():
                    def redo(i, cc2):
                        sl = pl.ds(i * _L, _L)
                        six, sp, _ = _decode(wb_[sl])
                        sp = sp + lax.broadcast(i * _L, (_L,))
                        _retry_max(acc1, six, plsc.load_gather(f1, [sp]))
                        _retry_max(acc2, six, plsc.load_gather(f2, [sp]))
                        return cc2

                    lax.fori_loop(0, _CHV, redo, 0)

                @pl.when(ci + 2 < _NCH)
                def _next():
                    _start(ci + 2, bufsets[bi])

            return cc

        lax.fori_loop(0, _NCH // 2, chunk_grp, 0)

        def wb_fix(i, cc):
            for u in range(2):
                sl = pl.ds((i * 2 + u) * _L, _L)
                v1 = acc1[sl]
                acc1[sl] = jnp.where(v1 == neginf, zero, v1)
                v2 = acc2[sl]
                acc2[sl] = jnp.where(v2 == neginf, zero, v2)
            return cc

        lax.fori_loop(0, _ACCV // 2, wb_fix, 0)
        pltpu.sync_copy(acc1, vmax_ref.at[pl.ds(r1 * _NV, _NV)])
        pltpu.sync_copy(acc2, vmax_ref.at[pl.ds(r2 * _NV, _NV)])
        return carry

    lax.fori_loop(0, 8, pair, 0)

    @pl.when(s % 8 == 0)
    def _counts():
        ones = _splat(1.0, jnp.float32)

        def ini(i, cc):
            acc2[pl.ds(i * _L, _L)] = zero
            return cc

        lax.fori_loop(0, _ACCV, ini, 0)

        def chunk(ci, cc):
            pltpu.sync_copy(wstr_ref.at[pl.ds(wbase + ci * _CH, _CH)],
                            wba.at[pl.ds(0, _CH)])

            def inner(i, c3):
                six = wba[pl.ds(i * _L, _L)] & m15
                plsc.addupdate_scatter(acc2, [six], ones)
                return c3

            lax.fori_loop(0, _CHV, inner, 0)
            return cc

        lax.fori_loop(0, _NCH, chunk, 0)

        def fix(i, cc):
            sl = pl.ds(i * _L, _L)
            acc2[sl] = jnp.maximum(acc2[sl], ones)
            return cc

        lax.fori_loop(0, _ACCV, fix, 0)
        pltpu.sync_copy(acc2, cnt_ref.at[pl.ds(b * _NV, _NV)])


_MESH = plsc.VectorSubcoreMesh(core_axis_name="c", subcore_axis_name="s")

_SC_CALL = pl.kernel(
    _sc_body,
    out_type=(
        jax.ShapeDtypeStruct((_IDX_LEN,), jnp.int32),
        jax.ShapeDtypeStruct((_B * _D * _NV,), jnp.float32),
        jax.ShapeDtypeStruct((_B * _NV,), jnp.float32),
        jax.ShapeDtypeStruct((_IDX_LEN,), jnp.int32),
    ),
    mesh=_MESH,
    compiler_params=pltpu.CompilerParams(needs_layout_passes=False),
    scratch_types=[
        pltpu.VMEM((_CH,), jnp.float32),
        pltpu.VMEM((_CH,), jnp.float32),
        pltpu.VMEM((_CH,), jnp.float32),
        pltpu.VMEM((_CH,), jnp.float32),
        pltpu.VMEM((_CH,), jnp.int32),
        pltpu.VMEM((_CH,), jnp.int32),
        pltpu.VMEM((_NV,), jnp.float32),
        pltpu.VMEM((_NV,), jnp.float32),
        pltpu.VMEM((2 * _L,), jnp.int32),
        pltpu.SemaphoreType.DMA,
        pltpu.SemaphoreType.DMA,
        pltpu.SemaphoreType.DMA,
        pltpu.SemaphoreType.DMA,
        pltpu.SemaphoreType.DMA,
        pltpu.SemaphoreType.DMA,
    ],
)


@jax.jit
def kernel(features, xyz_coords_for_voxelization):
    f = features.reshape(-1)
    xyz = xyz_coords_for_voxelization.reshape(-1)
    xyz = jnp.pad(xyz, (0, _IDX_LEN * 3 - xyz.shape[0]))
    idxp, vmax, cnt, _ = _SC_CALL(xyz, f)
    return (
        vmax.reshape(_B, _D, _G, _G, _G),
        idxp[: _B * _N].reshape(_B, _N),
        cnt.astype(jnp.int32).reshape(_B, 1, _NV),
    )


# speedup vs baseline: 2.5738x; 1.1395x over previous
import jax
import jax.numpy as jnp
from jax import lax
from jax.experimental import pallas as pl
from jax.experimental.pallas import tpu as pltpu
from jax.experimental.pallas import tpu_sc as plsc

_G = 32
_NV = _G * _G * _G
_B = 4
_D = 128
_N = 100000

_L = 16
_S1C = 3200
_S1T = 1600
_CH = 10000
_NCH = _N // _CH
_CHV = _CH // _L
_UNR = 5
_ACCV = _NV // _L


def _splat(val, dtype):
    return jnp.full((_L,), val, dtype)


def _sc_body(xyz_ref, feat_ref, idx_ref, vmax_ref, cnt_ref, wstr_ref,
             fb1a, fb1b, fb2a, fb2b, wba, wbb, acc1, acc2, tmp,
             sf1a, sf2a, swa, sf1b, sf2b, swb):
    c = lax.axis_index("c")
    s = lax.axis_index("s")
    lane = lax.iota(jnp.int32, _L)
    lanep = jnp.minimum(lane + 1, _splat(15, jnp.int32))
    lanem = jnp.maximum(lane - 1, _splat(0, jnp.int32))

    k31 = _splat(31, jnp.int32)
    k0 = _splat(0, jnp.int32)
    scale = _splat(32.0, jnp.float32)

    def _quant(u):
        ui = (u * scale).astype(jnp.int32)
        return jnp.minimum(jnp.maximum(ui, k0), k31)

    def _s1_chunk(off, npts):
        pltpu.sync_copy(xyz_ref.at[pl.ds(off * 3, npts * 3)],
                        fb1a.at[pl.ds(0, npts * 3)])

        def s1_body(g, carry):
            ix = g * 48 + lane * 3
            x = plsc.load_gather(fb1a, [ix])
            y = plsc.load_gather(fb1a, [ix + 1])
            z = plsc.load_gather(fb1a, [ix + 2])
            flat = (_quant(x) * _splat(_G * _G, jnp.int32)
                    + _quant(y) * _splat(_G, jnp.int32) + _quant(z))
            wba[pl.ds(g * _L, _L)] = flat
            sk, sperm = plsc.sort_key_val(flat, lane)
            tmp[pl.ds(0, _L)] = sk
            tmp[pl.ds(_L, _L)] = sperm
            skp = plsc.load_gather(tmp, [lanep])
            skm = plsc.load_gather(tmp, [lanem])
            spp = plsc.load_gather(tmp, [lanep + _L])
            spm = plsc.load_gather(tmp, [lanem + _L])
            nb = jnp.where(sk == skp, spp,
                           jnp.where(sk == skm, spm, sperm))
            word = sk + sperm * _splat(1 << 15, jnp.int32) \
                + nb * _splat(1 << 19, jnp.int32)
            wbb[pl.ds(g * _L, _L)] = word
            return carry

        lax.fori_loop(0, npts // _L, s1_body, 0)
        pltpu.sync_copy(wba.at[pl.ds(0, npts)], idx_ref.at[pl.ds(off, npts)])
        pltpu.sync_copy(wbb.at[pl.ds(0, npts)], wstr_ref.at[pl.ds(off, npts)])

    for t in range(3):
        _s1_chunk(c * 200000 + (s + 16 * t) * _S1C, _S1C)

    @pl.when(s < 14)
    def _s1_t3():
        _s1_chunk(c * 200000 + (s + 48) * _S1C, _S1C)

    @pl.when(s == 15)
    def _s1_tail():
        _s1_chunk(c * 200000 + 62 * _S1C, _S1T)

    plsc.subcore_barrier()

    b = c * 2 + s // 8
    dbase = (s % 8) * 16
    wbase = b * _N
    neginf = _splat(-jnp.inf, jnp.float32)
    zero = _splat(0.0, jnp.float32)
    false_v = jnp.zeros((_L,), jnp.bool_)
    m15 = _splat((1 << 15) - 1, jnp.int32)
    m4 = _splat(15, jnp.int32)

    def _decode(w, i):
        base = _splat(i * _L, jnp.int32)
        six = w & m15
        sp = ((w >> _splat(15, jnp.int32)) & m4) | base
        nb = ((w >> _splat(19, jnp.int32)) & m4) | base
        return six, sp, nb

    def _retry_max(accref, vidx, vals):
        cur = plsc.load_gather(accref, [vidx])
        plsc.store_scatter(accref, [vidx], jnp.maximum(cur, vals))
        chk = plsc.load_gather(accref, [vidx])
        nbad = jnp.sum((chk < vals).astype(jnp.int32))

        def fcond(nb_):
            return nb_ > 0

        def fbody(nb_):
            c2 = plsc.load_gather(accref, [vidx])
            plsc.store_scatter(accref, [vidx], jnp.maximum(c2, vals),
                               mask=c2 < vals)
            c4 = plsc.load_gather(accref, [vidx])
            return jnp.sum((c4 < vals).astype(jnp.int32))

        lax.while_loop(fcond, fbody, nbad)

    bufsets = ((fb1a, fb2a, (sf1a, sf2a, swa), wba),
               (fb1b, fb2b, (sf1b, sf2b, swb), wbb))

    def pair(p, carry):
        r1 = b * _D + dbase + 2 * p
        r2 = r1 + 1

        def _copies(ci, bufset):
            f1, f2, sems, wb_ = bufset
            return (
                pltpu.make_async_copy(
                    feat_ref.at[pl.ds(r1 * _N + ci * _CH, _CH)], f1, sems[0]),
                pltpu.make_async_copy(
                    feat_ref.at[pl.ds(r2 * _N + ci * _CH, _CH)], f2, sems[1]),
                pltpu.make_async_copy(
                    wstr_ref.at[pl.ds(wbase + ci * _CH, _CH)], wb_, sems[2]),
            )

        def _start(ci, bufset):
            for cp in _copies(ci, bufset):
                cp.start()

        def _wait(ci, bufset):
            for cp in _copies(ci, bufset):
                cp.wait()

        _start(0, bufsets[0])
        _start(1, bufsets[1])

        def ini(i, cc):
            for u in range(4):
                sl = pl.ds((i * 4 + u) * _L, _L)
                acc1[sl] = neginf
                acc2[sl] = neginf
            return cc

        lax.fori_loop(0, _ACCV // 4, ini, 0)

        def chunk_grp(g2, cc):
            for bi in range(2):
                f1, f2, sems, wb_ = bufsets[bi]
                ci = 2 * g2 + bi
                _wait(ci, bufsets[bi])

                def inner(i2, bad):
                    grp = []
                    for u in range(_UNR):
                        i = i2 * _UNR + u
                        w = wb_[pl.ds(i * _L, _L)]
                        six, sp, nb = _decode(w, i)
                        v1 = jnp.maximum(plsc.load_gather(f1, [sp]),
                                         plsc.load_gather(f1, [nb]))
                        v2 = jnp.maximum(plsc.load_gather(f2, [sp]),
                                         plsc.load_gather(f2, [nb]))
                        grp.append((six, v1, v2))
                    for six, v1, _v in grp:
                        c1 = plsc.load_gather(acc1, [six])
                        plsc.store_scatter(acc1, [six], jnp.maximum(c1, v1))
                    for six, _v, v2 in grp:
                        c2 = plsc.load_gather(acc2, [six])
                        plsc.store_scatter(acc2, [six], jnp.maximum(c2, v2))
                    for six, v1, v2 in grp:
                        w1 = plsc.load_gather(acc1, [six])
                        w2 = plsc.load_gather(acc2, [six])
                        bad = bad | (w1 < v1) | (w2 < v2)
                    return bad

                bad = lax.fori_loop(0, _CHV // _UNR, inner, false_v)
                nbad = jnp.sum(bad.astype(jnp.int32))

                @pl.when(nbad > 0)
                def _redo():
                    def redo(i, cc2):
                        six, sp, _ = _decode(wb_[pl.ds(i * _L, _L)], i)
                        _retry_max(acc1, six, plsc.load_gather(f1, [sp]))
                        _retry_max(acc2, six, plsc.load_gather(f2, [sp]))
                        return cc2

                    lax.fori_loop(0, _CHV, redo, 0)

                @pl.when(ci + 2 < _NCH)
                def _next():
                    _start(ci + 2, bufsets[bi])

            return cc

        lax.fori_loop(0, _NCH // 2, chunk_grp, 0)

        def wb_fix(i, cc):
            for u in range(2):
                sl = pl.ds((i * 2 + u) * _L, _L)
                v1 = acc1[sl]
                acc1[sl] = jnp.where(v1 == neginf, zero, v1)
                v2 = acc2[sl]
                acc2[sl] = jnp.where(v2 == neginf, zero, v2)
            return cc

        lax.fori_loop(0, _ACCV // 2, wb_fix, 0)
        pltpu.sync_copy(acc1, vmax_ref.at[pl.ds(r1 * _NV, _NV)])
        pltpu.sync_copy(acc2, vmax_ref.at[pl.ds(r2 * _NV, _NV)])
        return carry

    lax.fori_loop(0, 8, pair, 0)

    @pl.when(s % 8 == 0)
    def _counts():
        ones = _splat(1.0, jnp.float32)

        def ini(i, cc):
            acc2[pl.ds(i * _L, _L)] = zero
            return cc

        lax.fori_loop(0, _ACCV, ini, 0)

        def chunk(ci, cc):
            pltpu.sync_copy(wstr_ref.at[pl.ds(wbase + ci * _CH, _CH)],
                            wbb.at[pl.ds(0, _CH)])

            def inner(i, c3):
                six = wbb[pl.ds(i * _L, _L)] & m15
                plsc.addupdate_scatter(acc2, [six], ones)
                return c3

            lax.fori_loop(0, _CHV, inner, 0)
            return cc

        lax.fori_loop(0, _NCH, chunk, 0)

        for wnd in range(4):
            def cvt(i, cc):
                v = acc2[pl.ds((wnd * 512 + i) * _L, _L)]
                wba[pl.ds(i * _L, _L)] = \
                    jnp.maximum(v, ones).astype(jnp.int32)
                return cc

            lax.fori_loop(0, 512, cvt, 0)
            pltpu.sync_copy(wba.at[pl.ds(0, 8192)],
                            cnt_ref.at[pl.ds(b * _NV + wnd * 8192, 8192)])


_MESH = plsc.VectorSubcoreMesh(core_axis_name="c", subcore_axis_name="s")

_SC_CALL = pl.kernel(
    _sc_body,
    out_type=(
        jax.ShapeDtypeStruct((_B * _N,), jnp.int32),
        jax.ShapeDtypeStruct((_B * _D * _NV,), jnp.float32),
        jax.ShapeDtypeStruct((_B * _NV,), jnp.int32),
        jax.ShapeDtypeStruct((_B * _N,), jnp.int32),
    ),
    mesh=_MESH,
    compiler_params=pltpu.CompilerParams(needs_layout_passes=False),
    scratch_types=[
        pltpu.VMEM((_CH,), jnp.float32),
        pltpu.VMEM((_CH,), jnp.float32),
        pltpu.VMEM((_CH,), jnp.float32),
        pltpu.VMEM((_CH,), jnp.float32),
        pltpu.VMEM((_CH,), jnp.int32),
        pltpu.VMEM((_CH,), jnp.int32),
        pltpu.VMEM((_NV,), jnp.float32),
        pltpu.VMEM((_NV,), jnp.float32),
        pltpu.VMEM((2 * _L,), jnp.int32),
        pltpu.SemaphoreType.DMA,
        pltpu.SemaphoreType.DMA,
        pltpu.SemaphoreType.DMA,
        pltpu.SemaphoreType.DMA,
        pltpu.SemaphoreType.DMA,
        pltpu.SemaphoreType.DMA,
    ],
)


@jax.jit
def kernel(features, xyz_coords_for_voxelization):
    f = features.reshape(-1)
    xyz = xyz_coords_for_voxelization.reshape(-1)
    idxp, vmax, cnt, _ = _SC_CALL(xyz, f)
    return (
        vmax.reshape(_B, _D, _G, _G, _G),
        idxp.reshape(_B, _N),
        cnt.reshape(_B, 1, _NV),
    )


# speedup vs baseline: 2.7320x; 1.0615x over previous
import jax
import jax.numpy as jnp
from jax import lax
from jax.experimental import pallas as pl
from jax.experimental.pallas import tpu as pltpu
from jax.experimental.pallas import tpu_sc as plsc

_G = 32
_NV = _G * _G * _G
_B = 4
_D = 128
_N = 100000

_L = 16
_S1C = 640
_S1T = 320
_CH = 2000
_NCH = _N // _CH
_CHV = _CH // _L
_UNR = 5
_ACCV = _NV // _L


def _splat(val, dtype):
    return jnp.full((_L,), val, dtype)


def _sc_body(xyz_ref, feat_ref, idx_ref, vmax_ref, cnt_ref, wstr_ref,
             fb1a, fb1b, fb2a, fb2b, fb3a, fb3b, wba, wbb,
             acc1, acc2, acc3, tmp,
             sf1a, sf2a, sf3a, swa, sf1b, sf2b, sf3b, swb):
    c = lax.axis_index("c")
    s = lax.axis_index("s")
    lane = lax.iota(jnp.int32, _L)
    lanep = jnp.minimum(lane + 1, _splat(15, jnp.int32))
    lanem = jnp.maximum(lane - 1, _splat(0, jnp.int32))

    k31 = _splat(31, jnp.int32)
    k0 = _splat(0, jnp.int32)
    scale = _splat(32.0, jnp.float32)

    def _quant(u):
        ui = (u * scale).astype(jnp.int32)
        return jnp.minimum(jnp.maximum(ui, k0), k31)

    def _s1_chunk(off, npts):
        pltpu.sync_copy(xyz_ref.at[pl.ds(off * 3, npts * 3)],
                        fb1a.at[pl.ds(0, npts * 3)])

        def s1_body(g, carry):
            ix = g * 48 + lane * 3
            x = plsc.load_gather(fb1a, [ix])
            y = plsc.load_gather(fb1a, [ix + 1])
            z = plsc.load_gather(fb1a, [ix + 2])
            flat = (_quant(x) * _splat(_G * _G, jnp.int32)
                    + _quant(y) * _splat(_G, jnp.int32) + _quant(z))
            wba[pl.ds(g * _L, _L)] = flat
            sk, sperm = plsc.sort_key_val(flat, lane)
            tmp[pl.ds(0, _L)] = sk
            tmp[pl.ds(_L, _L)] = sperm
            skp = plsc.load_gather(tmp, [lanep])
            skm = plsc.load_gather(tmp, [lanem])
            spp = plsc.load_gather(tmp, [lanep + _L])
            spm = plsc.load_gather(tmp, [lanem + _L])
            nb = jnp.where(sk == skp, spp,
                           jnp.where(sk == skm, spm, sperm))
            word = sk + sperm * _splat(1 << 15, jnp.int32) \
                + nb * _splat(1 << 19, jnp.int32)
            wbb[pl.ds(g * _L, _L)] = word
            return carry

        lax.fori_loop(0, npts // _L, s1_body, 0)
        pltpu.sync_copy(wba.at[pl.ds(0, npts)], idx_ref.at[pl.ds(off, npts)])
        pltpu.sync_copy(wbb.at[pl.ds(0, npts)], wstr_ref.at[pl.ds(off, npts)])

    for t in range(19):
        _s1_chunk(c * 200000 + (s + 16 * t) * _S1C, _S1C)

    @pl.when(s < 8)
    def _s1_t19():
        _s1_chunk(c * 200000 + (s + 304) * _S1C, _S1C)

    @pl.when(s == 15)
    def _s1_tail():
        _s1_chunk(c * 200000 + 312 * _S1C, _S1T)

    plsc.subcore_barrier()

    b = c * 2 + s // 8
    dbase = (s % 8) * 16
    wbase = b * _N
    neginf = _splat(-jnp.inf, jnp.float32)
    zero = _splat(0.0, jnp.float32)
    false_v = jnp.zeros((_L,), jnp.bool_)
    m15 = _splat((1 << 15) - 1, jnp.int32)
    m4 = _splat(15, jnp.int32)

    def _decode(w, i):
        base = _splat(i * _L, jnp.int32)
        six = w & m15
        sp = ((w >> _splat(15, jnp.int32)) & m4) | base
        nb = ((w >> _splat(19, jnp.int32)) & m4) | base
        return six, sp, nb

    def _retry_max(accref, vidx, vals):
        cur = plsc.load_gather(accref, [vidx])
        plsc.store_scatter(accref, [vidx], jnp.maximum(cur, vals))
        chk = plsc.load_gather(accref, [vidx])
        nbad = jnp.sum((chk < vals).astype(jnp.int32))

        def fcond(nb_):
            return nb_ > 0

        def fbody(nb_):
            c2 = plsc.load_gather(accref, [vidx])
            plsc.store_scatter(accref, [vidx], jnp.maximum(c2, vals),
                               mask=c2 < vals)
            c4 = plsc.load_gather(accref, [vidx])
            return jnp.sum((c4 < vals).astype(jnp.int32))

        lax.while_loop(fcond, fbody, nbad)

    accs = (acc1, acc2, acc3)
    bufsets = (((fb1a, fb2a, fb3a), (sf1a, sf2a, sf3a, swa), wba),
               ((fb1b, fb2b, fb3b), (sf1b, sf2b, sf3b, swb), wbb))

    def _triple(rows):
        nr = len(rows)

        def _copies(ci, bufset):
            fbs, sems, wb_ = bufset
            cps = [pltpu.make_async_copy(
                feat_ref.at[pl.ds(rows[k] * _N + ci * _CH, _CH)],
                fbs[k], sems[k]) for k in range(nr)]
            cps.append(pltpu.make_async_copy(
                wstr_ref.at[pl.ds(wbase + ci * _CH, _CH)], wb_, sems[3]))
            return cps

        def _start(ci, bufset):
            for cp in _copies(ci, bufset):
                cp.start()

        def _wait(ci, bufset):
            for cp in _copies(ci, bufset):
                cp.wait()

        _start(0, bufsets[0])
        _start(1, bufsets[1])

        def ini(i, cc):
            for u in range(4):
                sl = pl.ds((i * 4 + u) * _L, _L)
                for k in range(nr):
                    accs[k][sl] = neginf
            return cc

        lax.fori_loop(0, _ACCV // 4, ini, 0)

        def chunk_grp(g2, cc):
            for bi in range(2):
                fbs, sems, wb_ = bufsets[bi]
                ci = 2 * g2 + bi
                _wait(ci, bufsets[bi])

                def inner(i2, bad):
                    grp = []
                    for u in range(_UNR):
                        i = i2 * _UNR + u
                        w = wb_[pl.ds(i * _L, _L)]
                        six, sp, nb = _decode(w, i)
                        vals = [jnp.maximum(plsc.load_gather(fbs[k], [sp]),
                                            plsc.load_gather(fbs[k], [nb]))
                                for k in range(nr)]
                        grp.append((six, vals))
                    for six, vals in grp:
                        for k in range(nr):
                            ck = plsc.load_gather(accs[k], [six])
                            plsc.store_scatter(accs[k], [six],
                                               jnp.maximum(ck, vals[k]))
                    for six, vals in grp:
                        for k in range(nr):
                            wk = plsc.load_gather(accs[k], [six])
                            bad = bad | (wk < vals[k])
                    return bad

                bad = lax.fori_loop(0, _CHV // _UNR, inner, false_v)
                nbad = jnp.sum(bad.astype(jnp.int32))

                @pl.when(nbad > 0)
                def _redo():
                    def redo(i, cc2):
                        six, sp, _ = _decode(wb_[pl.ds(i * _L, _L)], i)
                        for k in range(nr):
                            _retry_max(accs[k], six,
                                       plsc.load_gather(fbs[k], [sp]))
                        return cc2

                    lax.fori_loop(0, _CHV, redo, 0)

                @pl.when(ci + 2 < _NCH)
                def _next():
                    _start(ci + 2, bufsets[bi])

            return cc

        lax.fori_loop(0, _NCH // 2, chunk_grp, 0)

        def wb_fix(i, cc):
            for u in range(2):
                sl = pl.ds((i * 2 + u) * _L, _L)
                for k in range(nr):
                    vk = accs[k][sl]
                    accs[k][sl] = jnp.where(vk == neginf, zero, vk)
            return cc

        lax.fori_loop(0, _ACCV // 2, wb_fix, 0)
        for k in range(nr):
            pltpu.sync_copy(accs[k], vmax_ref.at[pl.ds(rows[k] * _NV, _NV)])

    def trip(q, carry):
        r1 = b * _D + dbase + 3 * q
        _triple((r1, r1 + 1, r1 + 2))
        return carry

    lax.fori_loop(0, 5, trip, 0)
    _triple((b * _D + dbase + 15,))

    @pl.when(s % 8 == 0)
    def _counts():
        ones = _splat(1.0, jnp.float32)

        def ini(i, cc):
            acc2[pl.ds(i * _L, _L)] = zero
            return cc

        lax.fori_loop(0, _ACCV, ini, 0)

        def chunk(ci, cc):
            pltpu.sync_copy(wstr_ref.at[pl.ds(wbase + ci * _CH, _CH)],
                            wbb.at[pl.ds(0, _CH)])

            def inner(i, c3):
                six = wbb[pl.ds(i * _L, _L)] & m15
                plsc.addupdate_scatter(acc2, [six], ones)
                return c3

            lax.fori_loop(0, _CHV, inner, 0)
            return cc

        lax.fori_loop(0, _NCH, chunk, 0)

        for wnd in range(32):
            def cvt(i, cc):
                v = acc2[pl.ds((wnd * 64 + i) * _L, _L)]
                wba[pl.ds(i * _L, _L)] = \
                    jnp.maximum(v, ones).astype(jnp.int32)
                return cc

            lax.fori_loop(0, 64, cvt, 0)
            pltpu.sync_copy(wba.at[pl.ds(0, 1024)],
                            cnt_ref.at[pl.ds(b * _NV + wnd * 1024, 1024)])


_MESH = plsc.VectorSubcoreMesh(core_axis_name="c", subcore_axis_name="s")

_SC_CALL = pl.kernel(
    _sc_body,
    out_type=(
        jax.ShapeDtypeStruct((_B * _N,), jnp.int32),
        jax.ShapeDtypeStruct((_B * _D * _NV,), jnp.float32),
        jax.ShapeDtypeStruct((_B * _NV,), jnp.int32),
        jax.ShapeDtypeStruct((_B * _N,), jnp.int32),
    ),
    mesh=_MESH,
    compiler_params=pltpu.CompilerParams(needs_layout_passes=False),
    scratch_types=[
        pltpu.VMEM((_CH,), jnp.float32),
        pltpu.VMEM((_CH,), jnp.float32),
        pltpu.VMEM((_CH,), jnp.float32),
        pltpu.VMEM((_CH,), jnp.float32),
        pltpu.VMEM((_CH,), jnp.float32),
        pltpu.VMEM((_CH,), jnp.float32),
        pltpu.VMEM((_CH,), jnp.int32),
        pltpu.VMEM((_CH,), jnp.int32),
        pltpu.VMEM((_NV,), jnp.float32),
        pltpu.VMEM((_NV,), jnp.float32),
        pltpu.VMEM((_NV,), jnp.float32),
        pltpu.VMEM((2 * _L,), jnp.int32),
        pltpu.SemaphoreType.DMA,
        pltpu.SemaphoreType.DMA,
        pltpu.SemaphoreType.DMA,
        pltpu.SemaphoreType.DMA,
        pltpu.SemaphoreType.DMA,
        pltpu.SemaphoreType.DMA,
        pltpu.SemaphoreType.DMA,
        pltpu.SemaphoreType.DMA,
    ],
)


@jax.jit
def kernel(features, xyz_coords_for_voxelization):
    f = features.reshape(-1)
    xyz = xyz_coords_for_voxelization.reshape(-1)
    idxp, vmax, cnt, _ = _SC_CALL(xyz, f)
    return (
        vmax.reshape(_B, _D, _G, _G, _G),
        idxp.reshape(_B, _N),
        cnt.reshape(_B, 1, _NV),
    )


# speedup vs baseline: 2.7752x; 1.0158x over previous
import jax
import jax.numpy as jnp
from jax import lax
from jax.experimental import pallas as pl
from jax.experimental.pallas import tpu as pltpu
from jax.experimental.pallas import tpu_sc as plsc

_G = 32
_NV = _G * _G * _G
_B = 4
_D = 128
_N = 100000

_L = 16
_S1C = 640
_S1T = 320
_CH = 2000
_NCH = _N // _CH
_CHV = _CH // _L
_UNR = 5
_ACCV = _NV // _L


def _splat(val, dtype):
    return jnp.full((_L,), val, dtype)


def _sc_body(xyz_ref, feat_ref, idx_ref, vmax_ref, cnt_ref, wstr_ref,
             fb1a, fb1b, fb2a, fb2b, fb3a, fb3b, wba, wbb,
             acc1, acc2, acc3, tmp,
             sf1a, sf2a, sf3a, swa, sf1b, sf2b, sf3b, swb):
    c = lax.axis_index("c")
    s = lax.axis_index("s")
    lane = lax.iota(jnp.int32, _L)
    lanep = jnp.minimum(lane + 1, _splat(15, jnp.int32))
    lanem = jnp.maximum(lane - 1, _splat(0, jnp.int32))

    k31 = _splat(31, jnp.int32)
    k0 = _splat(0, jnp.int32)
    scale = _splat(32.0, jnp.float32)

    def _quant(u):
        ui = (u * scale).astype(jnp.int32)
        return jnp.minimum(jnp.maximum(ui, k0), k31)

    def _s1_chunk(off, npts):
        pltpu.sync_copy(xyz_ref.at[pl.ds(off * 3, npts * 3)],
                        fb1a.at[pl.ds(0, npts * 3)])

        def s1_body(g, carry):
            ix = g * 48 + lane * 3
            x = plsc.load_gather(fb1a, [ix])
            y = plsc.load_gather(fb1a, [ix + 1])
            z = plsc.load_gather(fb1a, [ix + 2])
            flat = (_quant(x) * _splat(_G * _G, jnp.int32)
                    + _quant(y) * _splat(_G, jnp.int32) + _quant(z))
            wba[pl.ds(g * _L, _L)] = flat
            sk, sperm = plsc.sort_key_val(flat, lane)
            tmp[pl.ds(0, _L)] = sk
            tmp[pl.ds(_L, _L)] = sperm
            skp = plsc.load_gather(tmp, [lanep])
            skm = plsc.load_gather(tmp, [lanem])
            spp = plsc.load_gather(tmp, [lanep + _L])
            spm = plsc.load_gather(tmp, [lanem + _L])
            nb = jnp.where(sk == skp, spp,
                           jnp.where(sk == skm, spm, sperm))
            word = sk + sperm * _splat(1 << 15, jnp.int32) \
                + nb * _splat(1 << 19, jnp.int32)
            wbb[pl.ds(g * _L, _L)] = word
            return carry

        lax.fori_loop(0, npts // _L, s1_body, 0)
        pltpu.sync_copy(wba.at[pl.ds(0, npts)], idx_ref.at[pl.ds(off, npts)])
        pltpu.sync_copy(wbb.at[pl.ds(0, npts)], wstr_ref.at[pl.ds(off, npts)])

    for t in range(19):
        _s1_chunk(c * 200000 + (s + 16 * t) * _S1C, _S1C)

    @pl.when(s < 8)
    def _s1_t19():
        _s1_chunk(c * 200000 + (s + 304) * _S1C, _S1C)

    @pl.when(s == 15)
    def _s1_tail():
        _s1_chunk(c * 200000 + 312 * _S1C, _S1T)

    plsc.subcore_barrier()

    b = c * 2 + s // 8
    dbase = (s % 8) * 16
    wbase = b * _N
    neginf = _splat(-jnp.inf, jnp.float32)
    zero = _splat(0.0, jnp.float32)
    false_v = jnp.zeros((_L,), jnp.bool_)
    m15 = _splat((1 << 15) - 1, jnp.int32)
    m4 = _splat(15, jnp.int32)

    def _decode(w, i):
        base = _splat(i * _L, jnp.int32)
        six = w & m15
        sp = ((w >> _splat(15, jnp.int32)) & m4) | base
        nb = ((w >> _splat(19, jnp.int32)) & m4) | base
        return six, sp, nb

    def _retry_max(accref, vidx, vals):
        cur = plsc.load_gather(accref, [vidx])
        plsc.store_scatter(accref, [vidx], jnp.maximum(cur, vals))
        chk = plsc.load_gather(accref, [vidx])
        nbad = jnp.sum((chk < vals).astype(jnp.int32))

        def fcond(nb_):
            return nb_ > 0

        def fbody(nb_):
            c2 = plsc.load_gather(accref, [vidx])
            plsc.store_scatter(accref, [vidx], jnp.maximum(c2, vals),
                               mask=c2 < vals)
            c4 = plsc.load_gather(accref, [vidx])
            return jnp.sum((c4 < vals).astype(jnp.int32))

        lax.while_loop(fcond, fbody, nbad)

    accs = (acc1, acc2, acc3)
    bufsets = (((fb1a, fb2a, fb3a), (sf1a, sf2a, sf3a, swa), wba),
               ((fb1b, fb2b, fb3b), (sf1b, sf2b, sf3b, swb), wbb))

    def _triple(rows):
        nr = len(rows)

        def _copies(ci, bufset):
            fbs, sems, wb_ = bufset
            cps = [pltpu.make_async_copy(
                feat_ref.at[pl.ds(rows[k] * _N + ci * _CH, _CH)],
                fbs[k], sems[k]) for k in range(nr)]
            cps.append(pltpu.make_async_copy(
                wstr_ref.at[pl.ds(wbase + ci * _CH, _CH)], wb_, sems[3]))
            return cps

        def _start(ci, bufset):
            for cp in _copies(ci, bufset):
                cp.start()

        def _wait(ci, bufset):
            for cp in _copies(ci, bufset):
                cp.wait()

        _start(0, bufsets[0])
        _start(1, bufsets[1])

        def ini(i, cc):
            for u in range(4):
                sl = pl.ds((i * 4 + u) * _L, _L)
                for k in range(nr):
                    accs[k][sl] = neginf
            return cc

        lax.fori_loop(0, _ACCV // 4, ini, 0)

        def chunk_grp(g2, cc):
            for bi in range(2):
                fbs, sems, wb_ = bufsets[bi]
                ci = 2 * g2 + bi
                _wait(ci, bufsets[bi])

                def inner(i2, bad):
                    grp = []
                    for u in range(_UNR):
                        i = i2 * _UNR + u
                        w = wb_[pl.ds(i * _L, _L)]
                        six, sp, nb = _decode(w, i)
                        vals = [jnp.maximum(plsc.load_gather(fbs[k], [sp]),
                                            plsc.load_gather(fbs[k], [nb]))
                                for k in range(nr)]
                        grp.append((six, vals))
                    for six, vals in grp:
                        for k in range(nr):
                            ck = plsc.load_gather(accs[k], [six])
                            plsc.store_scatter(accs[k], [six],
                                               jnp.maximum(ck, vals[k]))
                    for six, vals in grp:
                        for k in range(nr):
                            wk = plsc.load_gather(accs[k], [six])
                            bad = bad | (wk < vals[k])
                    return bad

                bad = lax.fori_loop(0, _CHV // _UNR, inner, false_v)
                nbad = jnp.sum(bad.astype(jnp.int32))

                @pl.when(nbad > 0)
                def _redo():
                    def redo(i, cc2):
                        six, sp, _ = _decode(wb_[pl.ds(i * _L, _L)], i)
                        for k in range(nr):
                            _retry_max(accs[k], six,
                                       plsc.load_gather(fbs[k], [sp]))
                        return cc2

                    lax.fori_loop(0, _CHV, redo, 0)

                @pl.when(ci + 2 < _NCH)
                def _next():
                    _start(ci + 2, bufsets[bi])

            return cc

        lax.fori_loop(0, _NCH // 2, chunk_grp, 0)

        def wb_fix(i, cc):
            for u in range(2):
                sl = pl.ds((i * 2 + u) * _L, _L)
                for k in range(nr):
                    vk = accs[k][sl]
                    accs[k][sl] = jnp.where(vk == neginf, zero, vk)
            return cc

        lax.fori_loop(0, _ACCV // 2, wb_fix, 0)
        for k in range(nr):
            pltpu.sync_copy(accs[k], vmax_ref.at[pl.ds(rows[k] * _NV, _NV)])

    def trip(q, carry):
        r1 = b * _D + dbase + 3 * q
        _triple((r1, r1 + 1, r1 + 2))
        return carry

    lax.fori_loop(0, 5, trip, 0)
    _triple((b * _D + dbase + 15,))

    @pl.when(s % 8 == 0)
    def _counts():
        ones = _splat(1.0, jnp.float32)

        def ini(i, cc):
            acc2[pl.ds(i * _L, _L)] = zero
            return cc

        lax.fori_loop(0, _ACCV, ini, 0)

        cbufs = ((wba, sf1a), (wbb, sf1b))

        def _ccopy(ci, cb):
            buf, sem = cb
            return pltpu.make_async_copy(
                wstr_ref.at[pl.ds(wbase + ci * _CH, _CH)], buf, sem)

        _ccopy(0, cbufs[0]).start()
        _ccopy(1, cbufs[1]).start()

        def chunk(g2, cc):
            for bi in range(2):
                buf, sem = cbufs[bi]
                ci = 2 * g2 + bi
                _ccopy(ci, cbufs[bi]).wait()

                def inner(i, c3):
                    six = buf[pl.ds(i * _L, _L)] & m15
                    plsc.addupdate_scatter(acc2, [six], ones)
                    return c3

                lax.fori_loop(0, _CHV, inner, 0)

                @pl.when(ci + 2 < _NCH)
                def _cnext():
                    _ccopy(ci + 2, cbufs[bi]).start()

            return cc

        lax.fori_loop(0, _NCH // 2, chunk, 0)

        for wnd in range(32):
            def cvt(i, cc):
                v = acc2[pl.ds((wnd * 64 + i) * _L, _L)]
                wba[pl.ds(i * _L, _L)] = \
                    jnp.maximum(v, ones).astype(jnp.int32)
                return cc

            lax.fori_loop(0, 64, cvt, 0)
            pltpu.sync_copy(wba.at[pl.ds(0, 1024)],
                            cnt_ref.at[pl.ds(b * _NV + wnd * 1024, 1024)])


_MESH = plsc.VectorSubcoreMesh(core_axis_name="c", subcore_axis_name="s")

_SC_CALL = pl.kernel(
    _sc_body,
    out_type=(
        jax.ShapeDtypeStruct((_B * _N,), jnp.int32),
        jax.ShapeDtypeStruct((_B * _D * _NV,), jnp.float32),
        jax.ShapeDtypeStruct((_B * _NV,), jnp.int32),
        jax.ShapeDtypeStruct((_B * _N,), jnp.int32),
    ),
    mesh=_MESH,
    compiler_params=pltpu.CompilerParams(needs_layout_passes=False),
    scratch_types=[
        pltpu.VMEM((_CH,), jnp.float32),
        pltpu.VMEM((_CH,), jnp.float32),
        pltpu.VMEM((_CH,), jnp.float32),
        pltpu.VMEM((_CH,), jnp.float32),
        pltpu.VMEM((_CH,), jnp.float32),
        pltpu.VMEM((_CH,), jnp.float32),
        pltpu.VMEM((_CH,), jnp.int32),
        pltpu.VMEM((_CH,), jnp.int32),
        pltpu.VMEM((_NV,), jnp.float32),
        pltpu.VMEM((_NV,), jnp.float32),
        pltpu.VMEM((_NV,), jnp.float32),
        pltpu.VMEM((2 * _L,), jnp.int32),
        pltpu.SemaphoreType.DMA,
        pltpu.SemaphoreType.DMA,
        pltpu.SemaphoreType.DMA,
        pltpu.SemaphoreType.DMA,
        pltpu.SemaphoreType.DMA,
        pltpu.SemaphoreType.DMA,
        pltpu.SemaphoreType.DMA,
        pltpu.SemaphoreType.DMA,
    ],
)


@jax.jit
def kernel(features, xyz_coords_for_voxelization):
    f = features.reshape(-1)
    xyz = xyz_coords_for_voxelization.reshape(-1)
    idxp, vmax, cnt, _ = _SC_CALL(xyz, f)
    return (
        vmax.reshape(_B, _D, _G, _G, _G),
        idxp.reshape(_B, _N),
        cnt.reshape(_B, 1, _NV),
    )


# speedup vs baseline: 2.7863x; 1.0040x over previous
import jax
import jax.numpy as jnp
from jax import lax
from jax.experimental import pallas as pl
from jax.experimental.pallas import tpu as pltpu
from jax.experimental.pallas import tpu_sc as plsc

_G = 32
_NV = _G * _G * _G
_B = 4
_D = 128
_N = 100000

_L = 16
_S1C = 640
_S1T = 320
_CH = 2000
_NCH = _N // _CH
_CHV = _CH // _L
_UNR = 5
_ACCV = _NV // _L


def _splat(val, dtype):
    return jnp.full((_L,), val, dtype)


def _sc_body(xyz_ref, feat_ref, idx_ref, vmax_ref, cnt_ref, wstr_ref,
             fb1a, fb1b, fb2a, fb2b, fb3a, fb3b, wba, wbb,
             acc1, acc2, acc3, tmp,
             sf1a, sf2a, sf3a, swa, sf1b, sf2b, sf3b, swb):
    c = lax.axis_index("c")
    s = lax.axis_index("s")
    lane = lax.iota(jnp.int32, _L)
    lanep = jnp.minimum(lane + 1, _splat(15, jnp.int32))
    lanem = jnp.maximum(lane - 1, _splat(0, jnp.int32))

    k31 = _splat(31, jnp.int32)
    k0 = _splat(0, jnp.int32)
    scale = _splat(32.0, jnp.float32)

    def _quant(u):
        ui = (u * scale).astype(jnp.int32)
        return jnp.minimum(jnp.maximum(ui, k0), k31)

    def _s1_chunk(off, npts):
        pltpu.sync_copy(xyz_ref.at[pl.ds(off * 3, npts * 3)],
                        fb1a.at[pl.ds(0, npts * 3)])

        def s1_body(g, carry):
            ix = g * 48 + lane * 3
            x = plsc.load_gather(fb1a, [ix])
            y = plsc.load_gather(fb1a, [ix + 1])
            z = plsc.load_gather(fb1a, [ix + 2])
            flat = (_quant(x) * _splat(_G * _G, jnp.int32)
                    + _quant(y) * _splat(_G, jnp.int32) + _quant(z))
            wba[pl.ds(g * _L, _L)] = flat
            sk, sperm = plsc.sort_key_val(flat, lane)
            tmp[pl.ds(0, _L)] = sk
            tmp[pl.ds(_L, _L)] = sperm
            skp = plsc.load_gather(tmp, [lanep])
            skm = plsc.load_gather(tmp, [lanem])
            spp = plsc.load_gather(tmp, [lanep + _L])
            spm = plsc.load_gather(tmp, [lanem + _L])
            nb = jnp.where(sk == skp, spp,
                           jnp.where(sk == skm, spm, sperm))
            word = sk + sperm * _splat(1 << 15, jnp.int32) \
                + nb * _splat(1 << 19, jnp.int32)
            wbb[pl.ds(g * _L, _L)] = word
            return carry

        lax.fori_loop(0, npts // _L, s1_body, 0)
        pltpu.sync_copy(wba.at[pl.ds(0, npts)], idx_ref.at[pl.ds(off, npts)])
        pltpu.sync_copy(wbb.at[pl.ds(0, npts)], wstr_ref.at[pl.ds(off, npts)])

    for t in range(19):
        _s1_chunk(c * 200000 + (s + 16 * t) * _S1C, _S1C)

    @pl.when(s < 8)
    def _s1_t19():
        _s1_chunk(c * 200000 + (s + 304) * _S1C, _S1C)

    @pl.when(s == 15)
    def _s1_tail():
        _s1_chunk(c * 200000 + 312 * _S1C, _S1T)

    plsc.subcore_barrier()

    b = c * 2 + s // 8
    dbase = (s % 8) * 16
    wbase = b * _N
    neginf = _splat(-jnp.inf, jnp.float32)
    zero = _splat(0.0, jnp.float32)
    false_v = jnp.zeros((_L,), jnp.bool_)
    m15 = _splat((1 << 15) - 1, jnp.int32)
    m4 = _splat(15, jnp.int32)

    def _decode(w, i):
        base = _splat(i * _L, jnp.int32)
        six = w & m15
        sp = ((w >> _splat(15, jnp.int32)) & m4) | base
        nb = ((w >> _splat(19, jnp.int32)) & m4) | base
        return six, sp, nb

    def _retry_max(accref, vidx, vals):
        cur = plsc.load_gather(accref, [vidx])
        plsc.store_scatter(accref, [vidx], jnp.maximum(cur, vals))
        chk = plsc.load_gather(accref, [vidx])
        nbad = jnp.sum((chk < vals).astype(jnp.int32))

        def fcond(nb_):
            return nb_ > 0

        def fbody(nb_):
            c2 = plsc.load_gather(accref, [vidx])
            plsc.store_scatter(accref, [vidx], jnp.maximum(c2, vals),
                               mask=c2 < vals)
            c4 = plsc.load_gather(accref, [vidx])
            return jnp.sum((c4 < vals).astype(jnp.int32))

        lax.while_loop(fcond, fbody, nbad)

    accs = (acc1, acc2, acc3)
    bufsets = (((fb1a, fb2a, fb3a), (sf1a, sf2a, sf3a, swa), wba),
               ((fb1b, fb2b, fb3b), (sf1b, sf2b, sf3b, swb), wbb))

    def _triple(rows):
        nr = len(rows)

        def _copies(ci, bufset):
            fbs, sems, wb_ = bufset
            cps = [pltpu.make_async_copy(
                feat_ref.at[pl.ds(rows[k] * _N + ci * _CH, _CH)],
                fbs[k], sems[k]) for k in range(nr)]
            cps.append(pltpu.make_async_copy(
                wstr_ref.at[pl.ds(wbase + ci * _CH, _CH)], wb_, sems[3]))
            return cps

        def _start(ci, bufset):
            for cp in _copies(ci, bufset):
                cp.start()

        def _wait(ci, bufset):
            for cp in _copies(ci, bufset):
                cp.wait()

        _start(0, bufsets[0])
        _start(1, bufsets[1])

        def ini(i, cc):
            for u in range(4):
                sl = pl.ds((i * 4 + u) * _L, _L)
                for k in range(nr):
                    accs[k][sl] = neginf
            return cc

        lax.fori_loop(0, _ACCV // 4, ini, 0)

        def chunk_grp(g2, cc):
            for bi in range(2):
                fbs, sems, wb_ = bufsets[bi]
                ci = 2 * g2 + bi
                _wait(ci, bufsets[bi])

                def inner(i2, bad):
                    grp = []
                    for u in range(_UNR):
                        i = i2 * _UNR + u
                        w = wb_[pl.ds(i * _L, _L)]
                        six, sp, nb = _decode(w, i)
                        vals = [jnp.maximum(plsc.load_gather(fbs[k], [sp]),
                                            plsc.load_gather(fbs[k], [nb]))
                                for k in range(nr)]
                        grp.append((six, vals))
                    for six, vals in grp:
                        for k in range(nr):
                            ck = plsc.load_gather(accs[k], [six])
                            plsc.store_scatter(accs[k], [six],
                                               jnp.maximum(ck, vals[k]))
                    for six, vals in grp:
                        for k in range(nr):
                            wk = plsc.load_gather(accs[k], [six])
                            bad = bad | (wk < vals[k])
                    return bad

                bad = lax.fori_loop(0, _CHV // _UNR, inner, false_v)
                nbad = jnp.sum(bad.astype(jnp.int32))

                @pl.when(nbad > 0)
                def _redo():
                    def redo(i, cc2):
                        six, sp, _ = _decode(wb_[pl.ds(i * _L, _L)], i)
                        for k in range(nr):
                            _retry_max(accs[k], six,
                                       plsc.load_gather(fbs[k], [sp]))
                        return cc2

                    lax.fori_loop(0, _CHV, redo, 0)

                @pl.when(ci + 2 < _NCH)
                def _next():
                    _start(ci + 2, bufsets[bi])

            return cc

        lax.fori_loop(0, _NCH // 2, chunk_grp, 0)

        def wb_fix(i, cc):
            for u in range(2):
                sl = pl.ds((i * 2 + u) * _L, _L)
                for k in range(nr):
                    vk = accs[k][sl]
                    accs[k][sl] = jnp.where(vk == neginf, zero, vk)
            return cc

        lax.fori_loop(0, _ACCV // 2, wb_fix, 0)
        for k in range(nr):
            pltpu.sync_copy(accs[k], vmax_ref.at[pl.ds(rows[k] * _NV, _NV)])

    def trip(q, carry):
        r1 = b * _D + dbase + 3 * q
        _triple((r1, r1 + 1, r1 + 2))
        return carry

    lax.fori_loop(0, 4, trip, 0)
    _triple((b * _D + dbase + 12, b * _D + dbase + 13))
    _triple((b * _D + dbase + 14, b * _D + dbase + 15))

    @pl.when(s % 8 == 0)
    def _counts():
        ones = _splat(1.0, jnp.float32)

        def ini(i, cc):
            acc2[pl.ds(i * _L, _L)] = zero
            return cc

        lax.fori_loop(0, _ACCV, ini, 0)

        cbufs = ((wba, sf1a), (wbb, sf1b))

        def _ccopy(ci, cb):
            buf, sem = cb
            return pltpu.make_async_copy(
                wstr_ref.at[pl.ds(wbase + ci * _CH, _CH)], buf, sem)

        _ccopy(0, cbufs[0]).start()
        _ccopy(1, cbufs[1]).start()

        def chunk(g2, cc):
            for bi in range(2):
                buf, sem = cbufs[bi]
                ci = 2 * g2 + bi
                _ccopy(ci, cbufs[bi]).wait()

                def inner(i, c3):
                    six = buf[pl.ds(i * _L, _L)] & m15
                    plsc.addupdate_scatter(acc2, [six], ones)
                    return c3

                lax.fori_loop(0, _CHV, inner, 0)

                @pl.when(ci + 2 < _NCH)
                def _cnext():
                    _ccopy(ci + 2, cbufs[bi]).start()

            return cc

        lax.fori_loop(0, _NCH // 2, chunk, 0)

        for wnd in range(32):
            def cvt(i, cc):
                v = acc2[pl.ds((wnd * 64 + i) * _L, _L)]
                wba[pl.ds(i * _L, _L)] = \
                    jnp.maximum(v, ones).astype(jnp.int32)
                return cc

            lax.fori_loop(0, 64, cvt, 0)
            pltpu.sync_copy(wba.at[pl.ds(0, 1024)],
                            cnt_ref.at[pl.ds(b * _NV + wnd * 1024, 1024)])


_MESH = plsc.VectorSubcoreMesh(core_axis_name="c", subcore_axis_name="s")

_SC_CALL = pl.kernel(
    _sc_body,
    out_type=(
        jax.ShapeDtypeStruct((_B * _N,), jnp.int32),
        jax.ShapeDtypeStruct((_B * _D * _NV,), jnp.float32),
        jax.ShapeDtypeStruct((_B * _NV,), jnp.int32),
        jax.ShapeDtypeStruct((_B * _N,), jnp.int32),
    ),
    mesh=_MESH,
    compiler_params=pltpu.CompilerParams(needs_layout_passes=False),
    scratch_types=[
        pltpu.VMEM((_CH,), jnp.float32),
        pltpu.VMEM((_CH,), jnp.float32),
        pltpu.VMEM((_CH,), jnp.float32),
        pltpu.VMEM((_CH,), jnp.float32),
        pltpu.VMEM((_CH,), jnp.float32),
        pltpu.VMEM((_CH,), jnp.float32),
        pltpu.VMEM((_CH,), jnp.int32),
        pltpu.VMEM((_CH,), jnp.int32),
        pltpu.VMEM((_NV,), jnp.float32),
        pltpu.VMEM((_NV,), jnp.float32),
        pltpu.VMEM((_NV,), jnp.float32),
        pltpu.VMEM((2 * _L,), jnp.int32),
        pltpu.SemaphoreType.DMA,
        pltpu.SemaphoreType.DMA,
        pltpu.SemaphoreType.DMA,
        pltpu.SemaphoreType.DMA,
        pltpu.SemaphoreType.DMA,
        pltpu.SemaphoreType.DMA,
        pltpu.SemaphoreType.DMA,
        pltpu.SemaphoreType.DMA,
    ],
)


@jax.jit
def kernel(features, xyz_coords_for_voxelization):
    f = features.reshape(-1)
    xyz = xyz_coords_for_voxelization.reshape(-1)
    idxp, vmax, cnt, _ = _SC_CALL(xyz, f)
    return (
        vmax.reshape(_B, _D, _G, _G, _G),
        idxp.reshape(_B, _N),
        cnt.reshape(_B, 1, _NV),
    )
